# Initial kernel scaffold; baseline (speedup 1.0000x reference)
import jax
import jax.numpy as jnp
from jax.experimental import pallas as pl


def kernel(x, edge_index, edge_attr, cheb_W, cheb_b, enc_W, enc_b, nn_W1, nn_b1, nn_W2, nn_b2, root_W, conv_b, ln_g, ln_b, lin_W, lin_b):
    raise NotImplementedError("write your pallas kernel here")


# speedup vs baseline: 1.9027x; 1.9027x over previous
import functools

import jax
import jax.numpy as jnp
from jax import lax
from jax.experimental import pallas as pl
from jax.experimental.pallas import tpu as pltpu
from jax.experimental.pallas import tpu_sc as plsc

N = 10000
E = 160000
F_IN = 128
F_EDGE = 16
NH = 32
EH = 16
H3 = 32
K = 5
NUM_LAYERS = 4
NUM_CLASSES = 40

TRASH = N
NP = 10112
SROWS = NP // 16
NW = 32
CH = 128
E_PAD = 163840
EPW = E_PAD // NW
NCHUNK = EPW // CH

_MESH = plsc.VectorSubcoreMesh(core_axis_name="c", subcore_axis_name="s")
_SC_PARAMS = pltpu.CompilerParams(use_tc_tiling_on_sc=False)

_STRIPE_CHUNKS = ((0, 128), (128, 128), (256, 128), (384, 128), (512, 120))


def _zero_acc(zeros_hbm, rows, acc, s):
    pltpu.sync_copy(zeros_hbm, rows)
    for off, sz in _STRIPE_CHUNKS:
        pltpu.sync_copy(rows.at[pl.ds(0, sz)], acc.at[pl.ds(s * SROWS + off, sz)])


def _write_acc(out, rows, acc, c, s):
    for off, sz in _STRIPE_CHUNKS:
        pltpu.sync_copy(acc.at[pl.ds(s * SROWS + off, sz)], rows.at[pl.ds(0, sz)])
        pltpu.sync_copy(rows.at[pl.ds(0, sz)], out.at[c, pl.ds(s * SROWS + off, sz)])


def _sc_prop(f):
    @functools.partial(
        pl.kernel,
        out_type=jax.ShapeDtypeStruct((2, NP, f), jnp.float32),
        mesh=_MESH,
        compiler_params=_SC_PARAMS,
        scratch_types=[
            pltpu.VMEM((CH,), jnp.int32),
            pltpu.VMEM((CH,), jnp.int32),
            pltpu.VMEM((CH, f), jnp.float32),
            pltpu.VMEM_SHARED((NP, f), jnp.float32),
            pltpu.SemaphoreType.DMA,
        ],
    )
    def k(table, srcm, dstm, zeros_hbm, out, sidx, didx, rows, acc, sem):
        c = lax.axis_index("c")
        s = lax.axis_index("s")
        wid = c * 16 + s
        _zero_acc(zeros_hbm, rows, acc, s)
        plsc.subcore_barrier()

        def body(i, _):
            base = wid * EPW + i * CH
            pltpu.sync_copy(srcm.at[pl.ds(base, CH)], sidx)
            pltpu.sync_copy(dstm.at[pl.ds(base, CH)], didx)
            pltpu.async_copy(table.at[sidx], rows, sem).wait()
            pltpu.sync_copy(rows, acc.at[didx], add=True)
            return 0

        lax.fori_loop(0, NCHUNK, body, 0)
        plsc.subcore_barrier()
        _write_acc(out, rows, acc, c, s)

    return k


def _sc_deg():
    f = 16

    @functools.partial(
        pl.kernel,
        out_type=jax.ShapeDtypeStruct((2, NP, f), jnp.float32),
        mesh=_MESH,
        compiler_params=_SC_PARAMS,
        scratch_types=[
            pltpu.VMEM((CH,), jnp.int32),
            pltpu.VMEM((CH,), jnp.int32),
            pltpu.VMEM((CH, f), jnp.float32),
            pltpu.VMEM((CH, f), jnp.float32),
            pltpu.VMEM_SHARED((NP, f), jnp.float32),
        ],
    )
    def k(srcm, zeros_hbm, ones_hbm, out, sidx_a, sidx_b, rows, ones_v, acc):
        c = lax.axis_index("c")
        s = lax.axis_index("s")
        wid = c * 16 + s
        _zero_acc(zeros_hbm, rows, acc, s)
        pltpu.sync_copy(ones_hbm, ones_v)
        plsc.subcore_barrier()

        def body(i, _):
            for j, sidx in ((0, sidx_a), (1, sidx_b)):
                base = wid * EPW + (2 * i + j) * CH
                pltpu.sync_copy(srcm.at[pl.ds(base, CH)], sidx)
                pltpu.sync_copy(ones_v, acc.at[sidx], add=True)
            return 0

        lax.fori_loop(0, NCHUNK // 2, body, 0)
        plsc.subcore_barrier()
        _write_acc(out, rows, acc, c, s)

    return k


def _sc_gather(f):
    @functools.partial(
        pl.kernel,
        out_type=jax.ShapeDtypeStruct((E_PAD, f), jnp.float32),
        mesh=_MESH,
        compiler_params=_SC_PARAMS,
        scratch_types=[
            pltpu.VMEM((CH,), jnp.int32),
            pltpu.VMEM((CH, f), jnp.float32),
            pltpu.SemaphoreType.DMA,
        ],
    )
    def k(table, idx, out, sidx, rows, sem):
        c = lax.axis_index("c")
        s = lax.axis_index("s")
        wid = c * 16 + s

        def body(i, _):
            base = wid * EPW + i * CH
            pltpu.sync_copy(idx.at[pl.ds(base, CH)], sidx)
            pltpu.async_copy(table.at[sidx], rows, sem).wait()
            pltpu.sync_copy(rows, out.at[pl.ds(base, CH)])
            return 0

        lax.fori_loop(0, NCHUNK, body, 0)

    return k


def _sc_scatter(f):
    @functools.partial(
        pl.kernel,
        out_type=jax.ShapeDtypeStruct((2, NP, f), jnp.float32),
        mesh=_MESH,
        compiler_params=_SC_PARAMS,
        scratch_types=[
            pltpu.VMEM((CH,), jnp.int32),
            pltpu.VMEM((CH, f), jnp.float32),
            pltpu.VMEM_SHARED((NP, f), jnp.float32),
        ],
    )
    def k(vals, didx_hbm, zeros_hbm, out, didx, rows, acc):
        c = lax.axis_index("c")
        s = lax.axis_index("s")
        wid = c * 16 + s
        _zero_acc(zeros_hbm, rows, acc, s)
        plsc.subcore_barrier()

        def body(i, _):
            base = wid * EPW + i * CH
            pltpu.sync_copy(didx_hbm.at[pl.ds(base, CH)], didx)
            pltpu.sync_copy(vals.at[pl.ds(base, CH)], rows)
            pltpu.sync_copy(rows, acc.at[didx], add=True)
            return 0

        lax.fori_loop(0, NCHUNK, body, 0)
        plsc.subcore_barrier()
        _write_acc(out, rows, acc, c, s)

    return k


def _mask_prep(src2, dst2, a02):
    def body(src_ref, dst_ref, a0_ref, srcm_ref, dstm_ref, srcg_ref):
        src = src_ref[...]
        dst = dst_ref[...]
        sh = src.shape
        pos = (lax.broadcasted_iota(jnp.int32, sh, 0) * 128
               + lax.broadcasted_iota(jnp.int32, sh, 1))
        trash = N + 1 + pos // EPW
        mask = (a0_ref[...] == 0.0) & (src != dst)
        srcm_ref[...] = jnp.where(mask, src, trash)
        dstm_ref[...] = jnp.where(mask, dst, trash)
        srcg_ref[...] = jnp.where(src == TRASH, 0, src)

    e2 = E_PAD // 128
    return pl.pallas_call(
        body,
        out_shape=(
            jax.ShapeDtypeStruct((e2, 128), jnp.int32),
            jax.ShapeDtypeStruct((e2, 128), jnp.int32),
            jax.ShapeDtypeStruct((e2, 128), jnp.int32),
        ),
    )(src2, dst2, a02)


def _ea_proj(eap, enc_W, enc_b):
    BE = 8192

    def body(ea_ref, w_ref, b_ref, o_ref):
        o_ref[...] = jnp.dot(ea_ref[...], w_ref[...],
                             preferred_element_type=jnp.float32) + b_ref[...]

    return pl.pallas_call(
        body,
        grid=(E_PAD // BE,),
        in_specs=[
            pl.BlockSpec((BE, F_EDGE), lambda i: (i, 0)),
            pl.BlockSpec((F_EDGE, EH), lambda i: (0, 0)),
            pl.BlockSpec((EH,), lambda i: (0,)),
        ],
        out_specs=pl.BlockSpec((BE, EH), lambda i: (i, 0)),
        out_shape=jax.ShapeDtypeStruct((E_PAD, EH), jnp.float32),
    )(eap, enc_W, enc_b)


def _cheb_init(xp, degp, W0):
    def body(x_ref, d_ref, w_ref, disp_ref, y_ref, out_ref):
        d = d_ref[...]
        deg = d[0, :, 0] + d[1, :, 0]
        pos = deg > 0.0
        dis = jnp.where(pos, lax.rsqrt(jnp.where(pos, deg, 1.0)), 0.0)
        disp_ref[...] = dis
        x = x_ref[...]
        y_ref[...] = dis[:, None] * x
        out_ref[...] = jnp.dot(x[:N], w_ref[...],
                               preferred_element_type=jnp.float32)

    return pl.pallas_call(
        body,
        out_shape=(
            jax.ShapeDtypeStruct((NP,), jnp.float32),
            jax.ShapeDtypeStruct((NP, F_IN), jnp.float32),
            jax.ShapeDtypeStruct((N, NH), jnp.float32),
        ),
    )(xp, degp, W0)


def _cheb_step(sp, disp, tx_prev, out_acc, Wk, first, last):
    def body(s_ref, dis_ref, tp_ref, acc_ref, w_ref, tx_ref, y_ref, out_ref):
        sarr = s_ref[...]
        ssum = sarr[0] + sarr[1]
        dis = dis_ref[...]
        p = -dis[:, None] * ssum
        tx = p if first else 2.0 * p - tp_ref[...]
        tx_ref[...] = tx
        if not last:
            y_ref[...] = dis[:, None] * tx
        else:
            y_ref[...] = jnp.zeros_like(tx)
        out_ref[...] = acc_ref[...] + jnp.dot(
            tx[:N], w_ref[...], preferred_element_type=jnp.float32)

    return pl.pallas_call(
        body,
        out_shape=(
            jax.ShapeDtypeStruct((NP, F_IN), jnp.float32),
            jax.ShapeDtypeStruct((NP, F_IN), jnp.float32),
            jax.ShapeDtypeStruct((N, NH), jnp.float32),
        ),
    )(sp, disp, tx_prev, out_acc, Wk)


def _nn_gemm(ts, ea, c2aug):
    BE = 2048
    grid = (E_PAD // BE,)

    def body(ts_ref, ea_ref, c_ref, msg_ref):
        t = ts_ref[...]
        e = ea_ref[...]
        parts = [e[:, f:f + 1] * t for f in range(EH)] + [t]
        za = jnp.concatenate(parts, axis=1)
        msg_ref[...] = jnp.dot(za, c_ref[...],
                               preferred_element_type=jnp.float32)

    return pl.pallas_call(
        body,
        grid=grid,
        in_specs=[
            pl.BlockSpec((BE, NH), lambda i: (i, 0)),
            pl.BlockSpec((BE, EH), lambda i: (i, 0)),
            pl.BlockSpec((EH * NH + NH, NH), lambda i: (0, 0)),
        ],
        out_specs=pl.BlockSpec((BE, NH), lambda i: (i, 0)),
        out_shape=jax.ShapeDtypeStruct((E_PAD, NH), jnp.float32),
    )(ts, ea, c2aug)


def _node_update(h_prev, aggp, t, rootW, convb, lng, lnb, layer):
    last = layer == NUM_LAYERS - 1

    def body(*refs):
        if layer == 0:
            a_ref, t_ref, rw_ref, cb_ref, g_ref, b_ref = refs[:6]
            h_ref, tn_ref = refs[6:]
        else:
            hp_ref, a_ref, t_ref, rw_ref, cb_ref, g_ref, b_ref = refs[:7]
            h_ref, tn_ref = refs[7:]
        a = a_ref[...]
        agg = a[0, :N] + a[1, :N]
        t = t_ref[...]
        h = agg + jnp.dot(t, rw_ref[...],
                          preferred_element_type=jnp.float32) + cb_ref[...]
        if layer != 0:
            h = h + hp_ref[...]
        h_ref[...] = h
        m = jnp.mean(h, axis=-1, keepdims=True)
        v = jnp.mean((h - m) ** 2, axis=-1, keepdims=True)
        ln = (h - m) * lax.rsqrt(v + 1e-5) * g_ref[...] + b_ref[...]
        tn_ref[...] = jnp.maximum(ln, 0.0)

    args = ([] if layer == 0 else [h_prev]) + [aggp, t, rootW, convb, lng, lnb]
    return pl.pallas_call(
        body,
        out_shape=(
            jax.ShapeDtypeStruct((N, NH), jnp.float32),
            jax.ShapeDtypeStruct((N, NH), jnp.float32),
        ),
    )(*args)


def _head(t, lin_W, lin_b):
    def body(t_ref, w_ref, b_ref, o_ref):
        o_ref[...] = jnp.dot(t_ref[...], w_ref[...],
                             preferred_element_type=jnp.float32) + b_ref[...]

    return pl.pallas_call(
        body,
        out_shape=jax.ShapeDtypeStruct((N, NUM_CLASSES), jnp.float32),
    )(t, lin_W, lin_b)


def kernel(x, edge_index, edge_attr, cheb_W, cheb_b, enc_W, enc_b, nn_W1,
           nn_b1, nn_W2, nn_b2, root_W, conv_b, ln_g, ln_b, lin_W, lin_b):
    src = edge_index[0].astype(jnp.int32)
    dst = edge_index[1].astype(jnp.int32)
    pad = E_PAD - E
    srcp = jnp.pad(src, (0, pad), constant_values=TRASH)
    dstp = jnp.pad(dst, (0, pad), constant_values=N + 34)
    eap = jnp.pad(edge_attr, ((0, pad), (0, 0)))
    xp = jnp.pad(x, ((0, NP - N), (0, 0)))

    zeros128 = jnp.zeros((CH, F_IN), jnp.float32)
    zeros32 = jnp.zeros((CH, NH), jnp.float32)
    zeros16 = jnp.zeros((CH, 16), jnp.float32)
    ones16 = jnp.ones((CH, 16), jnp.float32)

    W2r = nn_W2.reshape(NUM_LAYERS, H3, NH, NH)
    C2 = jnp.einsum('lfc,lcio->lfio', nn_W1, W2r).reshape(NUM_LAYERS, EH * NH, NH)
    D = nn_b2.reshape(NUM_LAYERS, NH, NH) + jnp.einsum('lc,lcio->lio', nn_b1, W2r)
    C2aug = jnp.concatenate([C2, D], axis=1)

    e2 = E_PAD // 128
    srcm2, dstm2, srcg2 = _mask_prep(
        srcp.reshape(e2, 128), dstp.reshape(e2, 128),
        eap[:, 0].reshape(e2, 128))
    srcm = srcm2.reshape(E_PAD)
    dstm = dstm2.reshape(E_PAD)
    srcg = srcg2.reshape(E_PAD)
    ea = _ea_proj(eap, enc_W, enc_b)

    degp = _sc_deg()(srcm, zeros16, ones16)
    disp, y, out_acc = _cheb_init(xp, degp, cheb_W[0])

    prop128 = _sc_prop(F_IN)
    txs = [xp]
    for k in range(1, K):
        sp = prop128(y, srcm, dstm, zeros128)
        prev2 = txs[k - 2] if k >= 2 else xp
        tx, y, out_acc = _cheb_step(
            sp, disp, prev2, out_acc, cheb_W[k],
            first=(k == 1), last=(k == K - 1))
        txs.append(tx)

    h = None
    t = out_acc + cheb_b

    gather32 = _sc_gather(NH)
    scatter32 = _sc_scatter(NH)
    for l in range(NUM_LAYERS):
        ts = gather32(t, srcg)
        msg = _nn_gemm(ts, ea, C2aug[l])
        aggp = scatter32(msg, dstp, zeros32)
        g = ln_g[l + 1] if l < NUM_LAYERS - 1 else ln_g[0]
        b = ln_b[l + 1] if l < NUM_LAYERS - 1 else ln_b[0]
        h, t = _node_update(h, aggp, t, root_W[l], conv_b[l], g, b, l)

    return _head(t, lin_W, lin_b)


# speedup vs baseline: 2.1429x; 1.1262x over previous
import functools

import jax
import jax.numpy as jnp
from jax import lax
from jax.experimental import pallas as pl
from jax.experimental.pallas import tpu as pltpu
from jax.experimental.pallas import tpu_sc as plsc

N = 10000
E = 160000
F_IN = 128
F_EDGE = 16
NH = 32
EH = 16
H3 = 32
K = 5
NUM_LAYERS = 4
NUM_CLASSES = 40

TRASH = N
NP = 10112
SROWS = NP // 16
NW = 32
CH = 128
E_PAD = 163840
EPW = E_PAD // NW
NCHUNK = EPW // CH

_MESH = plsc.VectorSubcoreMesh(core_axis_name="c", subcore_axis_name="s")
_SC_PARAMS = pltpu.CompilerParams(use_tc_tiling_on_sc=False)

_STRIPE_CHUNKS = ((0, 128), (128, 128), (256, 128), (384, 128), (512, 120))


def _zero_acc(zeros_hbm, rows, acc, s):
    pltpu.sync_copy(zeros_hbm, rows)
    for off, sz in _STRIPE_CHUNKS:
        pltpu.sync_copy(rows.at[pl.ds(0, sz)], acc.at[pl.ds(s * SROWS + off, sz)])


def _write_acc(out, rows, acc, c, s):
    for off, sz in _STRIPE_CHUNKS:
        pltpu.sync_copy(acc.at[pl.ds(s * SROWS + off, sz)], rows.at[pl.ds(0, sz)])
        pltpu.sync_copy(rows.at[pl.ds(0, sz)], out.at[c, pl.ds(s * SROWS + off, sz)])


def _sc_prop(f):
    @functools.partial(
        pl.kernel,
        out_type=jax.ShapeDtypeStruct((2, NP, f), jnp.float32),
        mesh=_MESH,
        compiler_params=_SC_PARAMS,
        scratch_types=[
            pltpu.VMEM((NCHUNK, CH), jnp.int32),
            pltpu.VMEM((NCHUNK, CH), jnp.int32),
            pltpu.VMEM((CH, f), jnp.float32),
            pltpu.VMEM((CH, f), jnp.float32),
            pltpu.VMEM_SHARED((NP, f), jnp.float32),
            pltpu.SemaphoreType.DMA,
        ],
    )
    def k(table, srcm2, dstm2, zeros_hbm, out, sidx, didx, rows_a, rows_b, acc,
          sem):
        c = lax.axis_index("c")
        s = lax.axis_index("s")
        wid = c * 16 + s
        bufs = (rows_a, rows_b)
        pltpu.sync_copy(srcm2.at[pl.ds(wid * NCHUNK, NCHUNK)], sidx)
        pltpu.sync_copy(dstm2.at[pl.ds(wid * NCHUNK, NCHUNK)], didx)

        def fire(ch, b):
            pltpu.async_copy(table.at[sidx.at[ch]], bufs[b], sem)

        def drain(ch, b):
            pltpu.make_async_copy(table.at[sidx.at[ch]], bufs[b], sem).wait()

        def consume(ch, b):
            pltpu.sync_copy(bufs[b], acc.at[didx.at[ch]], add=True)

        fire(0, 0)
        _zero_acc(zeros_hbm, rows_b, acc, s)
        plsc.subcore_barrier()

        def body(i, _):
            c0 = 2 * i
            fire(c0 + 1, 1)
            drain(c0, 0)
            consume(c0, 0)
            fire(c0 + 2, 0)
            drain(c0 + 1, 1)
            consume(c0 + 1, 1)
            return 0

        lax.fori_loop(0, NCHUNK // 2 - 1, body, 0)
        cl = NCHUNK - 2
        fire(cl + 1, 1)
        drain(cl, 0)
        consume(cl, 0)
        drain(cl + 1, 1)
        consume(cl + 1, 1)
        plsc.subcore_barrier()
        _write_acc(out, rows_a, acc, c, s)

    return k


def _sc_deg():
    f = 16

    @functools.partial(
        pl.kernel,
        out_type=jax.ShapeDtypeStruct((2, NP, f), jnp.float32),
        mesh=_MESH,
        compiler_params=_SC_PARAMS,
        scratch_types=[
            pltpu.VMEM((NCHUNK, CH), jnp.int32),
            pltpu.VMEM((CH, f), jnp.float32),
            pltpu.VMEM((CH, f), jnp.float32),
            pltpu.VMEM_SHARED((NP, f), jnp.float32),
        ],
    )
    def k(srcm2, zeros_hbm, ones_hbm, out, sidx, rows, ones_v, acc):
        c = lax.axis_index("c")
        s = lax.axis_index("s")
        wid = c * 16 + s
        pltpu.sync_copy(srcm2.at[pl.ds(wid * NCHUNK, NCHUNK)], sidx)
        _zero_acc(zeros_hbm, rows, acc, s)
        pltpu.sync_copy(ones_hbm, ones_v)
        plsc.subcore_barrier()

        def body(i, _):
            pltpu.sync_copy(ones_v, acc.at[sidx.at[i]], add=True)
            return 0

        lax.fori_loop(0, NCHUNK, body, 0)
        plsc.subcore_barrier()
        _write_acc(out, rows, acc, c, s)

    return k


def _sc_gather(f):
    @functools.partial(
        pl.kernel,
        out_type=jax.ShapeDtypeStruct((E_PAD, f), jnp.float32),
        mesh=_MESH,
        compiler_params=_SC_PARAMS,
        scratch_types=[
            pltpu.VMEM((NCHUNK, CH), jnp.int32),
            pltpu.VMEM((CH, f), jnp.float32),
            pltpu.VMEM((CH, f), jnp.float32),
            pltpu.SemaphoreType.DMA,
        ],
    )
    def k(table, idx2, out, sidx, rows_a, rows_b, sem):
        c = lax.axis_index("c")
        s = lax.axis_index("s")
        wid = c * 16 + s
        bufs = (rows_a, rows_b)
        pltpu.sync_copy(idx2.at[pl.ds(wid * NCHUNK, NCHUNK)], sidx)

        def fire(ch, b):
            pltpu.async_copy(table.at[sidx.at[ch]], bufs[b], sem)

        def drain(ch, b):
            pltpu.make_async_copy(table.at[sidx.at[ch]], bufs[b], sem).wait()

        def consume(ch, b):
            pltpu.sync_copy(bufs[b], out.at[pl.ds(wid * EPW + ch * CH, CH)])

        fire(0, 0)

        def body(i, _):
            c0 = 2 * i
            fire(c0 + 1, 1)
            drain(c0, 0)
            consume(c0, 0)
            fire(c0 + 2, 0)
            drain(c0 + 1, 1)
            consume(c0 + 1, 1)
            return 0

        lax.fori_loop(0, NCHUNK // 2 - 1, body, 0)
        cl = NCHUNK - 2
        fire(cl + 1, 1)
        drain(cl, 0)
        consume(cl, 0)
        drain(cl + 1, 1)
        consume(cl + 1, 1)

    return k


def _sc_scatter(f):
    @functools.partial(
        pl.kernel,
        out_type=jax.ShapeDtypeStruct((2, NP, f), jnp.float32),
        mesh=_MESH,
        compiler_params=_SC_PARAMS,
        scratch_types=[
            pltpu.VMEM((NCHUNK, CH), jnp.int32),
            pltpu.VMEM((CH, f), jnp.float32),
            pltpu.VMEM((CH, f), jnp.float32),
            pltpu.VMEM_SHARED((NP, f), jnp.float32),
            pltpu.SemaphoreType.DMA,
        ],
    )
    def k(vals, didx2, zeros_hbm, out, didx, rows_a, rows_b, acc, sem):
        c = lax.axis_index("c")
        s = lax.axis_index("s")
        wid = c * 16 + s
        bufs = (rows_a, rows_b)
        pltpu.sync_copy(didx2.at[pl.ds(wid * NCHUNK, NCHUNK)], didx)

        def fire(ch, b):
            pltpu.async_copy(vals.at[pl.ds(wid * EPW + ch * CH, CH)], bufs[b], sem)

        def drain(ch, b):
            pltpu.make_async_copy(
                vals.at[pl.ds(wid * EPW + ch * CH, CH)], bufs[b], sem).wait()

        def consume(ch, b):
            pltpu.sync_copy(bufs[b], acc.at[didx.at[ch]], add=True)

        fire(0, 0)
        _zero_acc(zeros_hbm, rows_b, acc, s)
        plsc.subcore_barrier()

        def body(i, _):
            c0 = 2 * i
            fire(c0 + 1, 1)
            drain(c0, 0)
            consume(c0, 0)
            fire(c0 + 2, 0)
            drain(c0 + 1, 1)
            consume(c0 + 1, 1)
            return 0

        lax.fori_loop(0, NCHUNK // 2 - 1, body, 0)
        cl = NCHUNK - 2
        fire(cl + 1, 1)
        drain(cl, 0)
        consume(cl, 0)
        drain(cl + 1, 1)
        consume(cl + 1, 1)
        plsc.subcore_barrier()
        _write_acc(out, rows_a, acc, c, s)

    return k


def _mask_prep(src2, dst2, a02):
    def body(src_ref, dst_ref, a0_ref, srcm_ref, dstm_ref, srcg_ref):
        src = src_ref[...]
        dst = dst_ref[...]
        sh = src.shape
        pos = (lax.broadcasted_iota(jnp.int32, sh, 0) * 128
               + lax.broadcasted_iota(jnp.int32, sh, 1))
        trash = N + 1 + pos // EPW
        mask = (a0_ref[...] == 0.0) & (src != dst)
        srcm_ref[...] = jnp.where(mask, src, trash)
        dstm_ref[...] = jnp.where(mask, dst, trash)
        srcg_ref[...] = jnp.where(src == TRASH, 0, src)

    e2 = E_PAD // 128
    return pl.pallas_call(
        body,
        out_shape=(
            jax.ShapeDtypeStruct((e2, 128), jnp.int32),
            jax.ShapeDtypeStruct((e2, 128), jnp.int32),
            jax.ShapeDtypeStruct((e2, 128), jnp.int32),
        ),
    )(src2, dst2, a02)


def _ea_proj(eap, enc_W, enc_b):
    BE = 8192

    def body(ea_ref, w_ref, b_ref, o_ref):
        o_ref[...] = jnp.dot(ea_ref[...], w_ref[...],
                             preferred_element_type=jnp.float32) + b_ref[...]

    return pl.pallas_call(
        body,
        grid=(E_PAD // BE,),
        in_specs=[
            pl.BlockSpec((BE, F_EDGE), lambda i: (i, 0)),
            pl.BlockSpec((F_EDGE, EH), lambda i: (0, 0)),
            pl.BlockSpec((EH,), lambda i: (0,)),
        ],
        out_specs=pl.BlockSpec((BE, EH), lambda i: (i, 0)),
        out_shape=jax.ShapeDtypeStruct((E_PAD, EH), jnp.float32),
    )(eap, enc_W, enc_b)


def _cheb_init(xp, degp, W0):
    def body(x_ref, d_ref, w_ref, disp_ref, y_ref, out_ref):
        d = d_ref[...]
        deg = d[0, :, 0] + d[1, :, 0]
        pos = deg > 0.0
        dis = jnp.where(pos, lax.rsqrt(jnp.where(pos, deg, 1.0)), 0.0)
        disp_ref[...] = dis
        x = x_ref[...]
        y_ref[...] = dis[:, None] * x
        out_ref[...] = jnp.dot(x[:N], w_ref[...],
                               preferred_element_type=jnp.float32)

    return pl.pallas_call(
        body,
        out_shape=(
            jax.ShapeDtypeStruct((NP,), jnp.float32),
            jax.ShapeDtypeStruct((NP, F_IN), jnp.float32),
            jax.ShapeDtypeStruct((N, NH), jnp.float32),
        ),
    )(xp, degp, W0)


def _cheb_step(sp, disp, tx_prev, out_acc, Wk, first, last):
    def body(s_ref, dis_ref, tp_ref, acc_ref, w_ref, tx_ref, y_ref, out_ref):
        sarr = s_ref[...]
        ssum = sarr[0] + sarr[1]
        dis = dis_ref[...]
        p = -dis[:, None] * ssum
        tx = p if first else 2.0 * p - tp_ref[...]
        tx_ref[...] = tx
        if not last:
            y_ref[...] = dis[:, None] * tx
        else:
            y_ref[...] = jnp.zeros_like(tx)
        out_ref[...] = acc_ref[...] + jnp.dot(
            tx[:N], w_ref[...], preferred_element_type=jnp.float32)

    return pl.pallas_call(
        body,
        out_shape=(
            jax.ShapeDtypeStruct((NP, F_IN), jnp.float32),
            jax.ShapeDtypeStruct((NP, F_IN), jnp.float32),
            jax.ShapeDtypeStruct((N, NH), jnp.float32),
        ),
    )(sp, disp, tx_prev, out_acc, Wk)


def _nn_gemm(ts, ea, c2aug):
    BE = 2048
    grid = (E_PAD // BE,)

    def body(ts_ref, ea_ref, c_ref, msg_ref):
        t = ts_ref[...]
        e = ea_ref[...]
        parts = [e[:, f:f + 1] * t for f in range(EH)] + [t]
        za = jnp.concatenate(parts, axis=1)
        msg_ref[...] = jnp.dot(za, c_ref[...],
                               preferred_element_type=jnp.float32)

    return pl.pallas_call(
        body,
        grid=grid,
        in_specs=[
            pl.BlockSpec((BE, NH), lambda i: (i, 0)),
            pl.BlockSpec((BE, EH), lambda i: (i, 0)),
            pl.BlockSpec((EH * NH + NH, NH), lambda i: (0, 0)),
        ],
        out_specs=pl.BlockSpec((BE, NH), lambda i: (i, 0)),
        out_shape=jax.ShapeDtypeStruct((E_PAD, NH), jnp.float32),
    )(ts, ea, c2aug)


def _node_update(h_prev, aggp, t, rootW, convb, lng, lnb, layer):
    last = layer == NUM_LAYERS - 1

    def body(*refs):
        if layer == 0:
            a_ref, t_ref, rw_ref, cb_ref, g_ref, b_ref = refs[:6]
            h_ref, tn_ref = refs[6:]
        else:
            hp_ref, a_ref, t_ref, rw_ref, cb_ref, g_ref, b_ref = refs[:7]
            h_ref, tn_ref = refs[7:]
        a = a_ref[...]
        agg = a[0, :N] + a[1, :N]
        t = t_ref[...]
        h = agg + jnp.dot(t, rw_ref[...],
                          preferred_element_type=jnp.float32) + cb_ref[...]
        if layer != 0:
            h = h + hp_ref[...]
        h_ref[...] = h
        m = jnp.mean(h, axis=-1, keepdims=True)
        v = jnp.mean((h - m) ** 2, axis=-1, keepdims=True)
        ln = (h - m) * lax.rsqrt(v + 1e-5) * g_ref[...] + b_ref[...]
        tn_ref[...] = jnp.maximum(ln, 0.0)

    args = ([] if layer == 0 else [h_prev]) + [aggp, t, rootW, convb, lng, lnb]
    return pl.pallas_call(
        body,
        out_shape=(
            jax.ShapeDtypeStruct((N, NH), jnp.float32),
            jax.ShapeDtypeStruct((N, NH), jnp.float32),
        ),
    )(*args)


def _head(t, lin_W, lin_b):
    def body(t_ref, w_ref, b_ref, o_ref):
        o_ref[...] = jnp.dot(t_ref[...], w_ref[...],
                             preferred_element_type=jnp.float32) + b_ref[...]

    return pl.pallas_call(
        body,
        out_shape=jax.ShapeDtypeStruct((N, NUM_CLASSES), jnp.float32),
    )(t, lin_W, lin_b)


def kernel(x, edge_index, edge_attr, cheb_W, cheb_b, enc_W, enc_b, nn_W1,
           nn_b1, nn_W2, nn_b2, root_W, conv_b, ln_g, ln_b, lin_W, lin_b):
    src = edge_index[0].astype(jnp.int32)
    dst = edge_index[1].astype(jnp.int32)
    pad = E_PAD - E
    srcp = jnp.pad(src, (0, pad), constant_values=TRASH)
    dstp = jnp.pad(dst, (0, pad), constant_values=N + 34)
    eap = jnp.pad(edge_attr, ((0, pad), (0, 0)))
    xp = jnp.pad(x, ((0, NP - N), (0, 0)))

    zeros128 = jnp.zeros((CH, F_IN), jnp.float32)
    zeros32 = jnp.zeros((CH, NH), jnp.float32)
    zeros16 = jnp.zeros((CH, 16), jnp.float32)
    ones16 = jnp.ones((CH, 16), jnp.float32)

    W2r = nn_W2.reshape(NUM_LAYERS, H3, NH, NH)
    C2 = jnp.einsum('lfc,lcio->lfio', nn_W1, W2r).reshape(NUM_LAYERS, EH * NH, NH)
    D = nn_b2.reshape(NUM_LAYERS, NH, NH) + jnp.einsum('lc,lcio->lio', nn_b1, W2r)
    C2aug = jnp.concatenate([C2, D], axis=1)

    e2 = E_PAD // 128
    srcm2, dstm2, srcg2 = _mask_prep(
        srcp.reshape(e2, 128), dstp.reshape(e2, 128),
        eap[:, 0].reshape(e2, 128))
    dstp2 = dstp.reshape(e2, 128)
    ea = _ea_proj(eap, enc_W, enc_b)

    degp = _sc_deg()(srcm2, zeros16, ones16)
    disp, y, out_acc = _cheb_init(xp, degp, cheb_W[0])

    prop128 = _sc_prop(F_IN)
    txs = [xp]
    for k in range(1, K):
        sp = prop128(y, srcm2, dstm2, zeros128)
        prev2 = txs[k - 2] if k >= 2 else xp
        tx, y, out_acc = _cheb_step(
            sp, disp, prev2, out_acc, cheb_W[k],
            first=(k == 1), last=(k == K - 1))
        txs.append(tx)

    h = None
    t = out_acc + cheb_b

    gather32 = _sc_gather(NH)
    scatter32 = _sc_scatter(NH)
    for l in range(NUM_LAYERS):
        ts = gather32(t, srcg2)
        msg = _nn_gemm(ts, ea, C2aug[l])
        aggp = scatter32(msg, dstp2, zeros32)
        g = ln_g[l + 1] if l < NUM_LAYERS - 1 else ln_g[0]
        b = ln_b[l + 1] if l < NUM_LAYERS - 1 else ln_b[0]
        h, t = _node_update(h, aggp, t, root_W[l], conv_b[l], g, b, l)

    return _head(t, lin_W, lin_b)
